# Initial kernel scaffold; baseline (speedup 1.0000x reference)
import jax
import jax.numpy as jnp
from jax.experimental import pallas as pl


def kernel(output_boxes, num_boxes):
    raise NotImplementedError("write your pallas kernel here")


# speedup vs baseline: 1.3682x; 1.3682x over previous
import functools

import jax
import jax.numpy as jnp
from jax.experimental import pallas as pl
from jax.experimental.pallas import tpu as pltpu

_SCORE_THRESH = 0.1
_NMS_THRESH = 0.7
_PRE = 2048
_POST = 500
_POST_PAD = 512
_ROWS = 160
_LANES = 128
_NBLK = _PRE // 128

_DOT = functools.partial(
    jax.lax.dot_general,
    precision=jax.lax.Precision.HIGHEST,
    preferred_element_type=jnp.float32,
)


def _row_major_excl_prefix(m):
    r, l = m.shape
    incl = m
    sh = 1
    while sh < l:
        incl = incl + jnp.concatenate(
            [jnp.zeros((r, sh), jnp.int32), incl[:, : l - sh]], axis=1)
        sh *= 2
    row_tot = incl[:, l - 1 : l]
    rows_incl = row_tot
    sh = 1
    while sh < r:
        rows_incl = rows_incl + jnp.concatenate(
            [jnp.zeros((sh, 1), jnp.int32), rows_incl[: r - sh, :]], axis=0)
        sh *= 2
    rows_excl = rows_incl - row_tot
    return rows_excl + (incl - m)


def _nms_kernel(nb_ref, box_ref, out_ref, acc_ref, pos_ref, diag_ref):
    b = pl.program_id(0)
    nb = nb_ref[b]

    score = box_ref[0, 7, :, :]
    idx = (jax.lax.broadcasted_iota(jnp.int32, (_ROWS, _LANES), 0) * _LANES
           + jax.lax.broadcasted_iota(jnp.int32, (_ROWS, _LANES), 1))
    valid = (idx < nb) & (score >= _SCORE_THRESH)
    keys = jnp.where(valid, jax.lax.bitcast_convert_type(score, jnp.int32),
                     jnp.int32(-1))

    def bisect(_, carry):
        lo, hi = carry
        mid = (lo + hi) // 2
        cnt = jnp.sum(jnp.where(keys >= mid, 1, 0).astype(jnp.int32))
        big = cnt >= _PRE
        return (jnp.where(big, mid, lo), jnp.where(big, hi, mid))

    k_thr, _ = jax.lax.fori_loop(
        0, 31, bisect, (jnp.int32(0), jnp.int32(0x3F800000)))

    c_gt = jnp.sum(jnp.where(keys >= k_thr + 1, 1, 0).astype(jnp.int32))
    quota = _PRE - c_gt
    eq = (keys == k_thr) & valid
    eq_pfx = _row_major_excl_prefix(eq.astype(jnp.int32))
    sel = (keys > k_thr) | (eq & (eq_pfx < quota))
    sel_i = sel.astype(jnp.int32)
    n_sel = jnp.sum(sel_i)
    pos_ref[...] = jnp.where(sel, _row_major_excl_prefix(sel_i), jnp.int32(-1))

    acc_ref[...] = jnp.zeros((_PRE, 9), jnp.float32)
    slot = jax.lax.broadcasted_iota(jnp.int32, (_PRE, 1), 0)

    def gather_chunk(c, _):
        data = jnp.reshape(box_ref[0, :, pl.ds(c, 1), :], (9, _LANES))
        p = pos_ref[pl.ds(c, 1), :]
        onehot = (slot == p).astype(jnp.float32)
        acc_ref[...] += _DOT(onehot, data, (((1,), (1,)), ((), ())))
        return 0

    jax.lax.fori_loop(0, _ROWS, gather_chunk, 0)
    compact = acc_ref[...]

    k_col = compact[:, 7:8]
    k_row = k_col.T
    col_idx = jax.lax.broadcasted_iota(jnp.int32, (1, _PRE), 1)
    rank_parts = []
    for rb in range(_NBLK):
        lo = rb * 128
        krb = k_col[lo : lo + 128, :]
        row_idx = lo + jax.lax.broadcasted_iota(jnp.int32, (128, 1), 0)
        gt = k_row > krb
        eq2 = (k_row == krb) & (col_idx < row_idx)
        rank_parts.append(
            jnp.sum((gt | eq2).astype(jnp.int32), axis=1, keepdims=True))
    rank_col = jnp.concatenate(rank_parts, axis=0)
    rank_row = rank_col.T

    sorted_parts = []
    for rb in range(_NBLK):
        lo = rb * 128
        tgt = lo + jax.lax.broadcasted_iota(jnp.int32, (128, 1), 0)
        perm = (rank_row == tgt).astype(jnp.float32)
        sorted_parts.append(_DOT(perm, compact, (((1,), (0,)), ((), ()))))
    cand = jnp.concatenate(sorted_parts, axis=0)

    xc, yc = cand[:, 0:1], cand[:, 1:2]
    dxc, dyc = cand[:, 3:4], cand[:, 4:5]
    x1_c, x2_c = xc - dxc * 0.5, xc + dxc * 0.5
    y1_c, y2_c = yc - dyc * 0.5, yc + dyc * 0.5
    area_c = dxc * dyc
    x1_r, x2_r = x1_c.T, x2_c.T
    y1_r, y2_r = y1_c.T, y2_c.T
    area_r = area_c.T

    lane2k = jax.lax.broadcasted_iota(jnp.int32, (1, _PRE), 1)
    lane128 = jax.lax.broadcasted_iota(jnp.int32, (1, 128), 1)
    keep = (lane2k < n_sel).astype(jnp.float32)

    for blk in range(_NBLK):
        lo = blk * 128
        ix1 = jnp.maximum(x1_c[lo : lo + 128, :], x1_r)
        ix2 = jnp.minimum(x2_c[lo : lo + 128, :], x2_r)
        iy1 = jnp.maximum(y1_c[lo : lo + 128, :], y1_r)
        iy2 = jnp.minimum(y2_c[lo : lo + 128, :], y2_r)
        inter = (jnp.clip(ix2 - ix1, 0.0, None) *
                 jnp.clip(iy2 - iy1, 0.0, None))
        union = area_c[lo : lo + 128, :] + area_r - inter
        iou = inter / jnp.maximum(union, 1e-6)
        supf = (iou > _NMS_THRESH).astype(jnp.float32)

        diag_ref[...] = supf[:, lo : lo + 128]
        keep_blk = keep[:, lo : lo + 128]

        def intra(i, kb):
            row_i = diag_ref[pl.ds(i, 1), :]
            keep_i = jnp.sum(kb * (lane128 == i).astype(jnp.float32))
            sup = (row_i > 0.5) & (lane128 > i) & (keep_i > 0.5)
            return jnp.where(sup, 0.0, kb)

        keep_blk = jax.lax.fori_loop(0, 128, intra, keep_blk)

        sup_cnt = _DOT(keep_blk, supf, (((1,), (0,)), ((), ())))
        later = (sup_cnt > 0.5) & (lane2k >= lo + 128)
        parts = (([keep[:, :lo]] if lo else []) + [keep_blk]
                 + ([keep[:, lo + 128:]] if lo + 128 < _PRE else []))
        keep = jnp.concatenate(parts, axis=1) if len(parts) > 1 else keep_blk
        keep = jnp.where(later, 0.0, keep)

    incl = keep
    sh = 1
    while sh < _PRE:
        incl = incl + jnp.concatenate(
            [jnp.zeros((1, sh), jnp.float32), incl[:, : _PRE - sh]], axis=1)
        sh *= 2
    p2 = jnp.where(keep > 0.5, (incl - keep).astype(jnp.int32), jnp.int32(-1))
    n_kept = jnp.sum(keep).astype(jnp.int32)

    slot_out = jax.lax.broadcasted_iota(jnp.int32, (_POST_PAD, 1), 0)
    onehot_out = (slot_out == p2).astype(jnp.float32)
    out = _DOT(onehot_out, cand, (((1,), (0,)), ((), ())))
    label = out[:, 8:9] + (slot_out < n_kept).astype(jnp.float32)
    out_ref[0, :, :] = jnp.concatenate([out[:, :8], label], axis=1)


def kernel(output_boxes, num_boxes):
    bsz, n, _ = output_boxes.shape
    arr = output_boxes[:, :, jnp.array([0, 1, 2, 3, 4, 5, 6, 8, 7])]
    pad = _ROWS * _LANES - n
    arr = jnp.pad(arr, ((0, 0), (0, pad), (0, 0)))
    arr = jnp.transpose(arr, (0, 2, 1)).reshape(bsz, 9, _ROWS, _LANES)
    nb = num_boxes.astype(jnp.int32)

    padded = pl.pallas_call(
        _nms_kernel,
        grid=(bsz,),
        in_specs=[
            pl.BlockSpec(memory_space=pltpu.SMEM),
            pl.BlockSpec((1, 9, _ROWS, _LANES), lambda b: (b, 0, 0, 0)),
        ],
        out_specs=pl.BlockSpec((1, _POST_PAD, 9), lambda b: (b, 0, 0)),
        out_shape=jax.ShapeDtypeStruct((bsz, _POST_PAD, 9), jnp.float32),
        scratch_shapes=[
            pltpu.VMEM((_PRE, 9), jnp.float32),
            pltpu.VMEM((_ROWS, _LANES), jnp.int32),
            pltpu.VMEM((128, 128), jnp.float32),
        ],
    )(nb, arr)
    return padded[:, :_POST, :]


# speedup vs baseline: 2.8889x; 2.1115x over previous
import functools

import jax
import jax.numpy as jnp
from jax.experimental import pallas as pl
from jax.experimental.pallas import tpu as pltpu

_SCORE_THRESH = 0.1
_NMS_THRESH = 0.7
_PRE = 2048
_POST = 500
_POST_PAD = 512
_ROWS = 160
_LANES = 128
_NBLK = _PRE // 128

_DOT = functools.partial(
    jax.lax.dot_general,
    precision=jax.lax.Precision.HIGHEST,
    preferred_element_type=jnp.float32,
)


def _row_major_excl_prefix(m):
    r, l = m.shape
    incl = m
    sh = 1
    while sh < l:
        incl = incl + jnp.concatenate(
            [jnp.zeros((r, sh), jnp.int32), incl[:, : l - sh]], axis=1)
        sh *= 2
    row_tot = incl[:, l - 1 : l]
    rows_incl = row_tot
    sh = 1
    while sh < r:
        rows_incl = rows_incl + jnp.concatenate(
            [jnp.zeros((sh, 1), jnp.int32), rows_incl[: r - sh, :]], axis=0)
        sh *= 2
    rows_excl = rows_incl - row_tot
    return rows_excl + (incl - m)


def _nms_kernel(nb_ref, box_ref, out_ref, pos_ref):
    b = pl.program_id(0)
    nb = nb_ref[b]

    score = box_ref[0, 7, :, :]
    idx = (jax.lax.broadcasted_iota(jnp.int32, (_ROWS, _LANES), 0) * _LANES
           + jax.lax.broadcasted_iota(jnp.int32, (_ROWS, _LANES), 1))
    valid = (idx < nb) & (score >= _SCORE_THRESH)
    keys = jnp.where(valid, jax.lax.bitcast_convert_type(score, jnp.int32),
                     jnp.int32(-1))

    def bisect(_, carry):
        lo, hi = carry
        mid = (lo + hi) // 2
        cnt = jnp.sum(jnp.where(keys >= mid, 1, 0).astype(jnp.int32))
        big = cnt >= _PRE
        return (jnp.where(big, mid, lo), jnp.where(big, hi, mid))

    k_thr, _ = jax.lax.fori_loop(
        0, 31, bisect, (jnp.int32(0), jnp.int32(0x3F800000)))

    c_gt = jnp.sum(jnp.where(keys >= k_thr + 1, 1, 0).astype(jnp.int32))
    quota = _PRE - c_gt
    eq = (keys == k_thr) & valid
    eq_pfx = _row_major_excl_prefix(eq.astype(jnp.int32))
    sel = (keys > k_thr) | (eq & (eq_pfx < quota))
    sel_i = sel.astype(jnp.int32)
    n_sel = jnp.sum(sel_i)
    pos_ref[...] = jnp.where(sel, _row_major_excl_prefix(sel_i), jnp.int32(-1))

    slot = jax.lax.broadcasted_iota(jnp.int32, (_PRE, 1), 0)

    def gather_chunk(c, acc):
        data = jnp.reshape(box_ref[0, :, pl.ds(c, 1), :], (9, _LANES))
        p = pos_ref[pl.ds(c, 1), :]
        onehot = (slot == p).astype(jnp.float32)
        return acc + _DOT(data, onehot, (((1,), (1,)), ((), ())))

    compact = jax.lax.fori_loop(
        0, _ROWS, gather_chunk, jnp.zeros((9, _PRE), jnp.float32))

    k_row = compact[7:8, :]
    k_col = k_row.T
    col_idx = jax.lax.broadcasted_iota(jnp.int32, (1, _PRE), 1)
    rank_parts = []
    for rb in range(_NBLK):
        lo = rb * 128
        krb = k_col[lo : lo + 128, :]
        row_idx = lo + jax.lax.broadcasted_iota(jnp.int32, (128, 1), 0)
        gt = k_row > krb
        eq2 = (k_row == krb) & (col_idx < row_idx)
        rank_parts.append(
            jnp.sum((gt | eq2).astype(jnp.int32), axis=1, keepdims=True))
    rank_col = jnp.concatenate(rank_parts, axis=0)
    rank_row = rank_col.T

    sorted_parts = []
    for rb in range(_NBLK):
        lo = rb * 128
        tgt = lo + jax.lax.broadcasted_iota(jnp.int32, (128, 1), 0)
        perm = (rank_row == tgt).astype(jnp.float32)
        sorted_parts.append(_DOT(perm, compact, (((1,), (1,)), ((), ()))))
    cand = jnp.concatenate(sorted_parts, axis=0)

    xc, yc = cand[:, 0:1], cand[:, 1:2]
    dxc, dyc = cand[:, 3:4], cand[:, 4:5]
    x1_c, x2_c = xc - dxc * 0.5, xc + dxc * 0.5
    y1_c, y2_c = yc - dyc * 0.5, yc + dyc * 0.5
    area_c = dxc * dyc
    x1_r, x2_r = x1_c.T, x2_c.T
    y1_r, y2_r = y1_c.T, y2_c.T
    area_r = area_c.T

    lane2k = jax.lax.broadcasted_iota(jnp.int32, (1, _PRE), 1)
    ut_mask = (jax.lax.broadcasted_iota(jnp.int32, (128, 128), 1)
               > jax.lax.broadcasted_iota(jnp.int32, (128, 128), 0)
               ).astype(jnp.float32)
    keep = (lane2k < n_sel).astype(jnp.float32)

    for blk in range(_NBLK):
        lo = blk * 128
        ix1 = jnp.maximum(x1_c[lo : lo + 128, :], x1_r)
        ix2 = jnp.minimum(x2_c[lo : lo + 128, :], x2_r)
        iy1 = jnp.maximum(y1_c[lo : lo + 128, :], y1_r)
        iy2 = jnp.minimum(y2_c[lo : lo + 128, :], y2_r)
        inter = (jnp.clip(ix2 - ix1, 0.0, None) *
                 jnp.clip(iy2 - iy1, 0.0, None))
        union = area_c[lo : lo + 128, :] + area_r - inter
        iou = inter / jnp.maximum(union, 1e-6)
        supf = (iou > _NMS_THRESH).astype(jnp.float32)

        s_blk = supf[:, lo : lo + 128] * ut_mask
        valid_blk = keep[:, lo : lo + 128]

        def fix_cond(carry):
            return carry[1]

        def fix_body(carry):
            kb, _ = carry
            cnt = _DOT(kb, s_blk, (((1,), (0,)), ((), ())))
            new = valid_blk * (cnt <= 0.5).astype(jnp.float32)
            return (new, jnp.any(new != kb))

        keep_blk, _ = jax.lax.while_loop(
            fix_cond, fix_body, (valid_blk, jnp.bool_(True)))

        sup_cnt = _DOT(keep_blk, supf, (((1,), (0,)), ((), ())))
        later = (sup_cnt > 0.5) & (lane2k >= lo + 128)
        parts = (([keep[:, :lo]] if lo else []) + [keep_blk]
                 + ([keep[:, lo + 128:]] if lo + 128 < _PRE else []))
        keep = jnp.concatenate(parts, axis=1) if len(parts) > 1 else keep_blk
        keep = jnp.where(later, 0.0, keep)

    incl = keep
    sh = 1
    while sh < _PRE:
        incl = incl + jnp.concatenate(
            [jnp.zeros((1, sh), jnp.float32), incl[:, : _PRE - sh]], axis=1)
        sh *= 2
    p2 = jnp.where(keep > 0.5, (incl - keep).astype(jnp.int32), jnp.int32(-1))
    n_kept = jnp.sum(keep).astype(jnp.int32)

    slot_out = jax.lax.broadcasted_iota(jnp.int32, (_POST_PAD, 1), 0)
    onehot_out = (slot_out == p2).astype(jnp.float32)
    out = _DOT(onehot_out, cand, (((1,), (0,)), ((), ())))
    label = out[:, 8:9] + (slot_out < n_kept).astype(jnp.float32)
    out_ref[0, :, :] = jnp.concatenate([out[:, :8], label], axis=1)


def kernel(output_boxes, num_boxes):
    bsz, n, _ = output_boxes.shape
    arr = output_boxes[:, :, jnp.array([0, 1, 2, 3, 4, 5, 6, 8, 7])]
    pad = _ROWS * _LANES - n
    arr = jnp.pad(arr, ((0, 0), (0, pad), (0, 0)))
    arr = jnp.transpose(arr, (0, 2, 1)).reshape(bsz, 9, _ROWS, _LANES)
    nb = num_boxes.astype(jnp.int32)

    padded = pl.pallas_call(
        _nms_kernel,
        grid=(bsz,),
        in_specs=[
            pl.BlockSpec(memory_space=pltpu.SMEM),
            pl.BlockSpec((1, 9, _ROWS, _LANES), lambda b: (b, 0, 0, 0)),
        ],
        out_specs=pl.BlockSpec((1, _POST_PAD, 9), lambda b: (b, 0, 0)),
        out_shape=jax.ShapeDtypeStruct((bsz, _POST_PAD, 9), jnp.float32),
        scratch_shapes=[pltpu.VMEM((_ROWS, _LANES), jnp.int32)],
        compiler_params=pltpu.CompilerParams(
            dimension_semantics=("parallel",)),
    )(nb, arr)
    return padded[:, :_POST, :]


# speedup vs baseline: 3.4756x; 1.2031x over previous
import functools

import jax
import jax.numpy as jnp
from jax.experimental import pallas as pl
from jax.experimental.pallas import tpu as pltpu

_SCORE_THRESH = 0.1
_NMS_THRESH = 0.7
_PRE = 2048
_POST = 500
_POST_PAD = 512
_ROWS = 20
_LANES = 1024
_NBLK = _PRE // 128

_DOT = functools.partial(
    jax.lax.dot_general,
    precision=jax.lax.Precision.HIGHEST,
    preferred_element_type=jnp.float32,
)


def _row_major_excl_prefix(m):
    r, l = m.shape
    incl = m
    sh = 1
    while sh < l:
        incl = incl + jnp.concatenate(
            [jnp.zeros((r, sh), jnp.int32), incl[:, : l - sh]], axis=1)
        sh *= 2
    row_tot = incl[:, l - 1 : l]
    rows_incl = row_tot
    sh = 1
    while sh < r:
        rows_incl = rows_incl + jnp.concatenate(
            [jnp.zeros((sh, 1), jnp.int32), rows_incl[: r - sh, :]], axis=0)
        sh *= 2
    rows_excl = rows_incl - row_tot
    return rows_excl + (incl - m)


def _nms_kernel(nb_ref, box_ref, out_ref, pos_ref):
    b = pl.program_id(0)
    nb = nb_ref[b]

    score = box_ref[0, 7, :, :]
    idx = (jax.lax.broadcasted_iota(jnp.int32, (_ROWS, _LANES), 0) * _LANES
           + jax.lax.broadcasted_iota(jnp.int32, (_ROWS, _LANES), 1))
    valid = (idx < nb) & (score >= _SCORE_THRESH)
    keys = jnp.where(valid, jax.lax.bitcast_convert_type(score, jnp.int32),
                     jnp.int32(-1))

    def bisect(_, carry):
        lo, hi = carry
        mid = (lo + hi) // 2
        cnt = jnp.sum(jnp.where(keys >= mid, 1, 0).astype(jnp.int32))
        big = cnt >= _PRE
        return (jnp.where(big, mid, lo), jnp.where(big, hi, mid))

    k_thr, _ = jax.lax.fori_loop(
        0, 31, bisect, (jnp.int32(0), jnp.int32(0x3F800000)))

    c_gt = jnp.sum(jnp.where(keys >= k_thr + 1, 1, 0).astype(jnp.int32))
    quota = _PRE - c_gt
    eq = (keys == k_thr) & valid
    eq_pfx = _row_major_excl_prefix(eq.astype(jnp.int32))
    sel = (keys > k_thr) | (eq & (eq_pfx < quota))
    sel_i = sel.astype(jnp.int32)
    n_sel = jnp.sum(sel_i)
    pos_ref[...] = jnp.where(sel, _row_major_excl_prefix(sel_i), jnp.int32(-1))

    slot = jax.lax.broadcasted_iota(jnp.int32, (_PRE, 1), 0)

    def gather_chunk(c, acc):
        data = jnp.reshape(box_ref[0, :, pl.ds(c, 1), :], (9, _LANES))
        p = pos_ref[pl.ds(c, 1), :]
        onehot = (slot == p).astype(jnp.float32)
        return acc + _DOT(data, onehot, (((1,), (1,)), ((), ())))

    compact = jax.lax.fori_loop(
        0, _ROWS, gather_chunk, jnp.zeros((9, _PRE), jnp.float32))

    k_row = compact[7:8, :]
    k_col = k_row.T
    col_idx = jax.lax.broadcasted_iota(jnp.int32, (1, _PRE), 1)
    rank_parts = []
    for rb in range(_NBLK):
        lo = rb * 128
        krb = k_col[lo : lo + 128, :]
        row_idx = lo + jax.lax.broadcasted_iota(jnp.int32, (128, 1), 0)
        gt = k_row > krb
        eq2 = (k_row == krb) & (col_idx < row_idx)
        rank_parts.append(
            jnp.sum((gt | eq2).astype(jnp.int32), axis=1, keepdims=True))
    rank_col = jnp.concatenate(rank_parts, axis=0)
    rank_row = rank_col.T

    sorted_parts = []
    for rb in range(_NBLK):
        lo = rb * 128
        tgt = lo + jax.lax.broadcasted_iota(jnp.int32, (128, 1), 0)
        perm = (rank_row == tgt).astype(jnp.float32)
        sorted_parts.append(_DOT(perm, compact, (((1,), (1,)), ((), ()))))
    cand = jnp.concatenate(sorted_parts, axis=0)

    xc, yc = cand[:, 0:1], cand[:, 1:2]
    dxc, dyc = cand[:, 3:4], cand[:, 4:5]
    x1_c, x2_c = xc - dxc * 0.5, xc + dxc * 0.5
    y1_c, y2_c = yc - dyc * 0.5, yc + dyc * 0.5
    area_c = dxc * dyc
    x1_r, x2_r = x1_c.T, x2_c.T
    y1_r, y2_r = y1_c.T, y2_c.T
    area_r = area_c.T

    lane2k = jax.lax.broadcasted_iota(jnp.int32, (1, _PRE), 1)
    ut_mask = (jax.lax.broadcasted_iota(jnp.int32, (128, 128), 1)
               > jax.lax.broadcasted_iota(jnp.int32, (128, 128), 0)
               ).astype(jnp.float32)
    keep = (lane2k < n_sel).astype(jnp.float32)

    for blk in range(_NBLK):
        lo = blk * 128
        ix1 = jnp.maximum(x1_c[lo : lo + 128, :], x1_r[:, lo:])
        ix2 = jnp.minimum(x2_c[lo : lo + 128, :], x2_r[:, lo:])
        iy1 = jnp.maximum(y1_c[lo : lo + 128, :], y1_r[:, lo:])
        iy2 = jnp.minimum(y2_c[lo : lo + 128, :], y2_r[:, lo:])
        inter = (jnp.clip(ix2 - ix1, 0.0, None) *
                 jnp.clip(iy2 - iy1, 0.0, None))
        union = area_c[lo : lo + 128, :] + area_r[:, lo:] - inter
        iou = inter / jnp.maximum(union, 1e-6)
        supf = (iou > _NMS_THRESH).astype(jnp.float32)

        s_blk = supf[:, :128] * ut_mask
        valid_blk = keep[:, lo : lo + 128]

        def fix_cond(carry):
            return carry[1]

        def fix_body(carry):
            kb, _ = carry
            cnt = _DOT(kb, s_blk, (((1,), (0,)), ((), ())))
            new = valid_blk * (cnt <= 0.5).astype(jnp.float32)
            return (new, jnp.any(new != kb))

        keep_blk, _ = jax.lax.while_loop(
            fix_cond, fix_body, (valid_blk, jnp.bool_(True)))

        parts = ([keep[:, :lo]] if lo else []) + [keep_blk]
        if lo + 128 < _PRE:
            sup_cnt = _DOT(keep_blk, supf[:, 128:],
                           (((1,), (0,)), ((), ())))
            parts.append(jnp.where(sup_cnt > 0.5, 0.0, keep[:, lo + 128:]))
        keep = jnp.concatenate(parts, axis=1) if len(parts) > 1 else keep_blk

    incl = keep
    sh = 1
    while sh < _PRE:
        incl = incl + jnp.concatenate(
            [jnp.zeros((1, sh), jnp.float32), incl[:, : _PRE - sh]], axis=1)
        sh *= 2
    p2 = jnp.where(keep > 0.5, (incl - keep).astype(jnp.int32), jnp.int32(-1))
    n_kept = jnp.sum(keep).astype(jnp.int32)

    slot_out = jax.lax.broadcasted_iota(jnp.int32, (_POST_PAD, 1), 0)
    onehot_out = (slot_out == p2).astype(jnp.float32)
    out = _DOT(onehot_out, cand, (((1,), (0,)), ((), ())))
    label = out[:, 8:9] + (slot_out < n_kept).astype(jnp.float32)
    out_ref[0, :, :] = jnp.concatenate([out[:, :8], label], axis=1)


def kernel(output_boxes, num_boxes):
    bsz, n, _ = output_boxes.shape
    arr = output_boxes[:, :, jnp.array([0, 1, 2, 3, 4, 5, 6, 8, 7])]
    pad = _ROWS * _LANES - n
    arr = jnp.pad(arr, ((0, 0), (0, pad), (0, 0)))
    arr = jnp.transpose(arr, (0, 2, 1)).reshape(bsz, 9, _ROWS, _LANES)
    nb = num_boxes.astype(jnp.int32)

    padded = pl.pallas_call(
        _nms_kernel,
        grid=(bsz,),
        in_specs=[
            pl.BlockSpec(memory_space=pltpu.SMEM),
            pl.BlockSpec((1, 9, _ROWS, _LANES), lambda b: (b, 0, 0, 0)),
        ],
        out_specs=pl.BlockSpec((1, _POST_PAD, 9), lambda b: (b, 0, 0)),
        out_shape=jax.ShapeDtypeStruct((bsz, _POST_PAD, 9), jnp.float32),
        scratch_shapes=[pltpu.VMEM((_ROWS, _LANES), jnp.int32)],
        compiler_params=pltpu.CompilerParams(
            dimension_semantics=("parallel",)),
    )(nb, arr)
    return padded[:, :_POST, :]


# speedup vs baseline: 5.7548x; 1.6558x over previous
import functools

import jax
import jax.numpy as jnp
from jax.experimental import pallas as pl
from jax.experimental.pallas import tpu as pltpu

_SCORE_THRESH = 0.1
_NMS_THRESH = 0.7
_PRE = 2048
_POST = 500
_POST_PAD = 512
_ROWS = 160
_LANES = 128
_NBLK = _PRE // 128

_DOT = functools.partial(
    jax.lax.dot_general,
    precision=jax.lax.Precision.HIGHEST,
    preferred_element_type=jnp.float32,
)


def _row_major_excl_prefix(m):
    r, l = m.shape
    incl = m
    sh = 1
    while sh < l:
        incl = incl + jnp.concatenate(
            [jnp.zeros((r, sh), jnp.int32), incl[:, : l - sh]], axis=1)
        sh *= 2
    row_tot = incl[:, l - 1 : l]
    rows_incl = row_tot
    sh = 1
    while sh < r:
        rows_incl = rows_incl + jnp.concatenate(
            [jnp.zeros((sh, 1), jnp.int32), rows_incl[: r - sh, :]], axis=0)
        sh *= 2
    rows_excl = rows_incl - row_tot
    return rows_excl + (incl - m)


def _nms_kernel(nb_ref, box_ref, out_ref, pos_ref):
    b = pl.program_id(0)
    nb = nb_ref[b]

    score = box_ref[0, 7, :, :]
    idx = (jax.lax.broadcasted_iota(jnp.int32, (_ROWS, _LANES), 0) * _LANES
           + jax.lax.broadcasted_iota(jnp.int32, (_ROWS, _LANES), 1))
    valid = (idx < nb) & (score >= _SCORE_THRESH)
    keys = jnp.where(valid, jax.lax.bitcast_convert_type(score, jnp.int32),
                     jnp.int32(-1))

    def bisect(_, carry):
        lo, hi = carry
        mid = (lo + hi) // 2
        cnt = jnp.sum(jnp.where(keys >= mid, 1, 0).astype(jnp.int32))
        big = cnt >= _PRE
        return (jnp.where(big, mid, lo), jnp.where(big, hi, mid))

    k_thr, _ = jax.lax.fori_loop(
        0, 31, bisect, (jnp.int32(0), jnp.int32(0x3F800000)))

    c_gt = jnp.sum(jnp.where(keys >= k_thr + 1, 1, 0).astype(jnp.int32))
    quota = _PRE - c_gt
    eq = (keys == k_thr) & valid
    eq_pfx = _row_major_excl_prefix(eq.astype(jnp.int32))
    sel = (keys > k_thr) | (eq & (eq_pfx < quota))
    sel_i = sel.astype(jnp.int32)
    n_sel = jnp.sum(sel_i)
    pos_ref[...] = jnp.where(sel, _row_major_excl_prefix(sel_i), jnp.int32(-1))

    slot = jax.lax.broadcasted_iota(jnp.int32, (2 * 128, 1), 0)

    def gather_chunk(c, carry):
        acc, base = carry
        data = jnp.reshape(box_ref[0, :, pl.ds(c, 1), :], (9, _LANES))
        p = pos_ref[pl.ds(c, 1), :]
        ba = base // 128
        onehot = (slot == (p - ba * 128)).astype(jnp.float32)
        contrib = _DOT(data, onehot, (((1,), (1,)), ((), ())))
        c0, c1 = contrib[:, :128], contrib[:, 128:]
        zero = jnp.zeros((9, 128), jnp.float32)
        placed = jnp.concatenate(
            [jnp.where(ba == t, c0, jnp.where(ba + 1 == t, c1, zero))
             for t in range(_NBLK)], axis=1)
        n_chunk = jnp.sum((p >= 0).astype(jnp.int32))
        return (acc + placed, base + n_chunk)

    compact, _ = jax.lax.fori_loop(
        0, _ROWS, gather_chunk,
        (jnp.zeros((9, _PRE), jnp.float32), jnp.int32(0)))

    k_row = compact[7:8, :]
    k_col = k_row.T
    col_idx = jax.lax.broadcasted_iota(jnp.int32, (1, _PRE), 1)
    iota128c = jax.lax.broadcasted_iota(jnp.int32, (128, 1), 0)
    rank_parts = []
    for rb in range(_NBLK):
        lo = rb * 128
        krb = k_col[lo : lo + 128, :]
        row_idx = lo + iota128c
        cnt = jnp.zeros((128, 1), jnp.int32)
        for kt in range(_NBLK):
            cs = slice(kt * 128, kt * 128 + 128)
            m = (k_row[:, cs] > krb) | ((k_row[:, cs] == krb)
                                        & (col_idx[:, cs] < row_idx))
            cnt = cnt + jnp.sum(m.astype(jnp.int32), axis=1, keepdims=True)
        rank_parts.append(cnt)
    rank_col = jnp.concatenate(rank_parts, axis=0)
    rank_row = rank_col.T

    sorted_parts = []
    for rb in range(_NBLK):
        lo = rb * 128
        tgt = lo + iota128c
        acc9 = jnp.zeros((128, 9), jnp.float32)
        for kt in range(_NBLK):
            cs = slice(kt * 128, kt * 128 + 128)
            perm = (rank_row[:, cs] == tgt).astype(jnp.float32)
            acc9 = acc9 + _DOT(perm, compact[:, cs], (((1,), (1,)), ((), ())))
        sorted_parts.append(acc9)
    cand = jnp.concatenate(sorted_parts, axis=0)

    xc, yc = cand[:, 0:1], cand[:, 1:2]
    dxc, dyc = cand[:, 3:4], cand[:, 4:5]
    x1_c, x2_c = xc - dxc * 0.5, xc + dxc * 0.5
    y1_c, y2_c = yc - dyc * 0.5, yc + dyc * 0.5
    area_c = dxc * dyc
    x1_r, x2_r = x1_c.T, x2_c.T
    y1_r, y2_r = y1_c.T, y2_c.T
    area_r = area_c.T

    lane2k = jax.lax.broadcasted_iota(jnp.int32, (1, _PRE), 1)
    ut_mask = (jax.lax.broadcasted_iota(jnp.int32, (128, 128), 1)
               > jax.lax.broadcasted_iota(jnp.int32, (128, 128), 0)
               ).astype(jnp.float32)
    keep = (lane2k < n_sel).astype(jnp.float32)

    def sup_tile(lo, jc):
        rs, cs = slice(lo, lo + 128), slice(jc, jc + 128)
        ix1 = jnp.maximum(x1_c[rs, :], x1_r[:, cs])
        ix2 = jnp.minimum(x2_c[rs, :], x2_r[:, cs])
        iy1 = jnp.maximum(y1_c[rs, :], y1_r[:, cs])
        iy2 = jnp.minimum(y2_c[rs, :], y2_r[:, cs])
        inter = (jnp.clip(ix2 - ix1, 0.0, None) *
                 jnp.clip(iy2 - iy1, 0.0, None))
        union = area_c[rs, :] + area_r[:, cs] - inter
        iou = inter / jnp.maximum(union, 1e-6)
        return (iou > _NMS_THRESH).astype(jnp.float32)

    for blk in range(_NBLK):
        lo = blk * 128

        s_blk = sup_tile(lo, lo) * ut_mask
        valid_blk = keep[:, lo : lo + 128]

        def fix_cond(carry):
            return carry[1]

        def fix_body(carry):
            kb, _ = carry
            cnt = _DOT(kb, s_blk, (((1,), (0,)), ((), ())))
            new = valid_blk * (cnt <= 0.5).astype(jnp.float32)
            return (new, jnp.any(new != kb))

        keep_blk, _ = jax.lax.while_loop(
            fix_cond, fix_body, (valid_blk, jnp.bool_(True)))

        parts = ([keep[:, :lo]] if lo else []) + [keep_blk]
        for jb in range(blk + 1, _NBLK):
            jc = jb * 128
            sup_cnt = _DOT(keep_blk, sup_tile(lo, jc),
                           (((1,), (0,)), ((), ())))
            parts.append(
                jnp.where(sup_cnt > 0.5, 0.0, keep[:, jc : jc + 128]))
        keep = jnp.concatenate(parts, axis=1) if len(parts) > 1 else keep_blk

    incl = keep
    sh = 1
    while sh < _PRE:
        incl = incl + jnp.concatenate(
            [jnp.zeros((1, sh), jnp.float32), incl[:, : _PRE - sh]], axis=1)
        sh *= 2
    p2 = jnp.where(keep > 0.5, (incl - keep).astype(jnp.int32), jnp.int32(-1))
    n_kept = jnp.sum(keep).astype(jnp.int32)

    slot_out = jax.lax.broadcasted_iota(jnp.int32, (_POST_PAD, 1), 0)
    out = jnp.zeros((_POST_PAD, 9), jnp.float32)
    for kt in range(_NBLK):
        cs = slice(kt * 128, kt * 128 + 128)
        oh = (slot_out == p2[:, cs]).astype(jnp.float32)
        out = out + _DOT(oh, cand[cs, :], (((1,), (0,)), ((), ())))
    label = out[:, 8:9] + (slot_out < n_kept).astype(jnp.float32)
    out_ref[0, :, :] = jnp.concatenate([out[:, :8], label], axis=1)


def kernel(output_boxes, num_boxes):
    bsz, n, _ = output_boxes.shape
    arr = output_boxes[:, :, jnp.array([0, 1, 2, 3, 4, 5, 6, 8, 7])]
    pad = _ROWS * _LANES - n
    arr = jnp.pad(arr, ((0, 0), (0, pad), (0, 0)))
    arr = jnp.transpose(arr, (0, 2, 1)).reshape(bsz, 9, _ROWS, _LANES)
    nb = num_boxes.astype(jnp.int32)

    padded = pl.pallas_call(
        _nms_kernel,
        grid=(bsz,),
        in_specs=[
            pl.BlockSpec(memory_space=pltpu.SMEM),
            pl.BlockSpec((1, 9, _ROWS, _LANES), lambda b: (b, 0, 0, 0)),
        ],
        out_specs=pl.BlockSpec((1, _POST_PAD, 9), lambda b: (b, 0, 0)),
        out_shape=jax.ShapeDtypeStruct((bsz, _POST_PAD, 9), jnp.float32),
        scratch_shapes=[pltpu.VMEM((_ROWS, _LANES), jnp.int32)],
        compiler_params=pltpu.CompilerParams(
            dimension_semantics=("parallel",)),
    )(nb, arr)
    return padded[:, :_POST, :]


# speedup vs baseline: 5.9266x; 1.0298x over previous
import functools

import jax
import jax.numpy as jnp
from jax.experimental import pallas as pl
from jax.experimental.pallas import tpu as pltpu

_SCORE_THRESH = 0.1
_NMS_THRESH = 0.7
_PRE = 2048
_POST = 500
_POST_PAD = 512
_ROWS = 160
_LANES = 128
_NBLK = _PRE // 128

_DOT = functools.partial(
    jax.lax.dot_general,
    precision=jax.lax.Precision.HIGHEST,
    preferred_element_type=jnp.float32,
)


def _row_major_excl_prefix(m):
    r, l = m.shape
    incl = m
    sh = 1
    while sh < l:
        incl = incl + jnp.concatenate(
            [jnp.zeros((r, sh), jnp.int32), incl[:, : l - sh]], axis=1)
        sh *= 2
    row_tot = incl[:, l - 1 : l]
    rows_incl = row_tot
    sh = 1
    while sh < r:
        rows_incl = rows_incl + jnp.concatenate(
            [jnp.zeros((sh, 1), jnp.int32), rows_incl[: r - sh, :]], axis=0)
        sh *= 2
    rows_excl = rows_incl - row_tot
    return rows_excl + (incl - m)


def _nms_kernel(nb_ref, box_ref, out_ref, pos_ref, acc_ref):
    b = pl.program_id(0)
    nb = nb_ref[b]

    score = box_ref[0, 7, :, :]
    idx = (jax.lax.broadcasted_iota(jnp.int32, (_ROWS, _LANES), 0) * _LANES
           + jax.lax.broadcasted_iota(jnp.int32, (_ROWS, _LANES), 1))
    valid = (idx < nb) & (score >= _SCORE_THRESH)
    keys = jnp.where(valid, jax.lax.bitcast_convert_type(score, jnp.int32),
                     jnp.int32(-1))

    def bisect(_, carry):
        lo, hi = carry
        mid = (lo + hi) // 2
        cnt = jnp.sum(jnp.where(keys >= mid, 1, 0).astype(jnp.int32))
        big = cnt >= _PRE
        return (jnp.where(big, mid, lo), jnp.where(big, hi, mid))

    k_thr, _ = jax.lax.fori_loop(
        0, 31, bisect, (jnp.int32(0), jnp.int32(0x3F800000)))

    c_gt = jnp.sum(jnp.where(keys >= k_thr + 1, 1, 0).astype(jnp.int32))
    quota = _PRE - c_gt
    eq = (keys == k_thr) & valid
    eq_pfx = _row_major_excl_prefix(eq.astype(jnp.int32))
    sel = (keys > k_thr) | (eq & (eq_pfx < quota))
    sel_i = sel.astype(jnp.int32)
    n_sel = jnp.sum(sel_i)
    pos_ref[...] = jnp.where(sel, _row_major_excl_prefix(sel_i), jnp.int32(-1))

    slot = jax.lax.broadcasted_iota(jnp.int32, (2 * 128, 1), 0)
    acc_ref[...] = jnp.zeros((9, _PRE + 256), jnp.float32)

    def gather_chunk(c, base):
        data = jnp.reshape(box_ref[0, :, pl.ds(c, 1), :], (9, _LANES))
        p = pos_ref[pl.ds(c, 1), :]
        ba = base // 128
        onehot = (slot == (p - ba * 128)).astype(jnp.float32)
        contrib = _DOT(data, onehot, (((1,), (1,)), ((), ())))
        acc_ref[:, pl.ds(ba * 128, 256)] += contrib
        return base + jnp.sum((p >= 0).astype(jnp.int32))

    jax.lax.fori_loop(0, _ROWS, gather_chunk, jnp.int32(0))
    compact = acc_ref[:, : _PRE]

    k_row = compact[7:8, :]
    k_col = k_row.T
    col_idx = jax.lax.broadcasted_iota(jnp.int32, (1, _PRE), 1)
    iota128c = jax.lax.broadcasted_iota(jnp.int32, (128, 1), 0)
    rank_parts = []
    for rb in range(_NBLK):
        lo = rb * 128
        krb = k_col[lo : lo + 128, :]
        row_idx = lo + iota128c
        cnt = jnp.zeros((128, 1), jnp.int32)
        for kt in range(_NBLK):
            cs = slice(kt * 128, kt * 128 + 128)
            m = (k_row[:, cs] > krb) | ((k_row[:, cs] == krb)
                                        & (col_idx[:, cs] < row_idx))
            cnt = cnt + jnp.sum(m.astype(jnp.int32), axis=1, keepdims=True)
        rank_parts.append(cnt)
    rank_col = jnp.concatenate(rank_parts, axis=0)
    rank_row = rank_col.T

    sorted_parts = []
    for rb in range(_NBLK):
        lo = rb * 128
        tgt = lo + iota128c
        acc9 = jnp.zeros((128, 9), jnp.float32)
        for kt in range(_NBLK):
            cs = slice(kt * 128, kt * 128 + 128)
            perm = (rank_row[:, cs] == tgt).astype(jnp.float32)
            acc9 = acc9 + _DOT(perm, compact[:, cs], (((1,), (1,)), ((), ())))
        sorted_parts.append(acc9)
    cand = jnp.concatenate(sorted_parts, axis=0)

    xc, yc = cand[:, 0:1], cand[:, 1:2]
    dxc, dyc = cand[:, 3:4], cand[:, 4:5]
    x1_c, x2_c = xc - dxc * 0.5, xc + dxc * 0.5
    y1_c, y2_c = yc - dyc * 0.5, yc + dyc * 0.5
    area_c = dxc * dyc
    x1_r, x2_r = x1_c.T, x2_c.T
    y1_r, y2_r = y1_c.T, y2_c.T
    area_r = area_c.T

    lane2k = jax.lax.broadcasted_iota(jnp.int32, (1, _PRE), 1)
    ut_mask = (jax.lax.broadcasted_iota(jnp.int32, (128, 128), 1)
               > jax.lax.broadcasted_iota(jnp.int32, (128, 128), 0)
               ).astype(jnp.float32)
    keep = (lane2k < n_sel).astype(jnp.float32)

    def sup_tile(lo, jc):
        rs, cs = slice(lo, lo + 128), slice(jc, jc + 128)
        ix1 = jnp.maximum(x1_c[rs, :], x1_r[:, cs])
        ix2 = jnp.minimum(x2_c[rs, :], x2_r[:, cs])
        iy1 = jnp.maximum(y1_c[rs, :], y1_r[:, cs])
        iy2 = jnp.minimum(y2_c[rs, :], y2_r[:, cs])
        inter = (jnp.clip(ix2 - ix1, 0.0, None) *
                 jnp.clip(iy2 - iy1, 0.0, None))
        union = area_c[rs, :] + area_r[:, cs] - inter
        iou = inter / jnp.maximum(union, 1e-6)
        return (iou > _NMS_THRESH).astype(jnp.float32)

    for blk in range(_NBLK):
        lo = blk * 128

        s_blk = sup_tile(lo, lo) * ut_mask
        valid_blk = keep[:, lo : lo + 128]

        def fix_cond(carry):
            return carry[1]

        def fix_body(carry):
            kb, _ = carry
            cnt = _DOT(kb, s_blk, (((1,), (0,)), ((), ())))
            new = valid_blk * (cnt <= 0.5).astype(jnp.float32)
            return (new, jnp.any(new != kb))

        keep_blk, _ = jax.lax.while_loop(
            fix_cond, fix_body, (valid_blk, jnp.bool_(True)))

        parts = ([keep[:, :lo]] if lo else []) + [keep_blk]
        for jb in range(blk + 1, _NBLK):
            jc = jb * 128
            sup_cnt = _DOT(keep_blk, sup_tile(lo, jc),
                           (((1,), (0,)), ((), ())))
            parts.append(
                jnp.where(sup_cnt > 0.5, 0.0, keep[:, jc : jc + 128]))
        keep = jnp.concatenate(parts, axis=1) if len(parts) > 1 else keep_blk

    incl = keep
    sh = 1
    while sh < _PRE:
        incl = incl + jnp.concatenate(
            [jnp.zeros((1, sh), jnp.float32), incl[:, : _PRE - sh]], axis=1)
        sh *= 2
    p2 = jnp.where(keep > 0.5, (incl - keep).astype(jnp.int32), jnp.int32(-1))
    n_kept = jnp.sum(keep).astype(jnp.int32)

    slot_out = jax.lax.broadcasted_iota(jnp.int32, (_POST_PAD, 1), 0)
    out = jnp.zeros((_POST_PAD, 9), jnp.float32)
    for kt in range(_NBLK):
        cs = slice(kt * 128, kt * 128 + 128)
        oh = (slot_out == p2[:, cs]).astype(jnp.float32)
        out = out + _DOT(oh, cand[cs, :], (((1,), (0,)), ((), ())))
    label = out[:, 8:9] + (slot_out < n_kept).astype(jnp.float32)
    out_ref[0, :, :] = jnp.concatenate([out[:, :8], label], axis=1)


def kernel(output_boxes, num_boxes):
    bsz, n, _ = output_boxes.shape
    arr = output_boxes[:, :, jnp.array([0, 1, 2, 3, 4, 5, 6, 8, 7])]
    pad = _ROWS * _LANES - n
    arr = jnp.pad(arr, ((0, 0), (0, pad), (0, 0)))
    arr = jnp.transpose(arr, (0, 2, 1)).reshape(bsz, 9, _ROWS, _LANES)
    nb = num_boxes.astype(jnp.int32)

    padded = pl.pallas_call(
        _nms_kernel,
        grid=(bsz,),
        in_specs=[
            pl.BlockSpec(memory_space=pltpu.SMEM),
            pl.BlockSpec((1, 9, _ROWS, _LANES), lambda b: (b, 0, 0, 0)),
        ],
        out_specs=pl.BlockSpec((1, _POST_PAD, 9), lambda b: (b, 0, 0)),
        out_shape=jax.ShapeDtypeStruct((bsz, _POST_PAD, 9), jnp.float32),
        scratch_shapes=[
            pltpu.VMEM((_ROWS, _LANES), jnp.int32),
            pltpu.VMEM((9, _PRE + 256), jnp.float32),
        ],
        compiler_params=pltpu.CompilerParams(
            dimension_semantics=("parallel",)),
    )(nb, arr)
    return padded[:, :_POST, :]


# speedup vs baseline: 6.5438x; 1.1041x over previous
import functools

import jax
import jax.numpy as jnp
from jax.experimental import pallas as pl
from jax.experimental.pallas import tpu as pltpu

_SCORE_THRESH = 0.1
_NMS_THRESH = 0.7
_PRE = 2048
_POST = 500
_POST_PAD = 512
_ROWS = 160
_LANES = 128
_NBLK = _PRE // 128

_DOT = functools.partial(
    jax.lax.dot_general,
    precision=jax.lax.Precision.HIGHEST,
    preferred_element_type=jnp.float32,
)


def _row_major_excl_prefix(m):
    r, l = m.shape
    incl = m
    sh = 1
    while sh < l:
        incl = incl + jnp.concatenate(
            [jnp.zeros((r, sh), jnp.int32), incl[:, : l - sh]], axis=1)
        sh *= 2
    row_tot = incl[:, l - 1 : l]
    rows_incl = row_tot
    sh = 1
    while sh < r:
        rows_incl = rows_incl + jnp.concatenate(
            [jnp.zeros((sh, 1), jnp.int32), rows_incl[: r - sh, :]], axis=0)
        sh *= 2
    rows_excl = rows_incl - row_tot
    return rows_excl + (incl - m)


def _nms_kernel(nb_ref, box_ref, out_ref, pos_ref, acc_ref):
    b = pl.program_id(0)
    nb = nb_ref[b]

    score = box_ref[0, 7, :, :]
    idx = (jax.lax.broadcasted_iota(jnp.int32, (_ROWS, _LANES), 0) * _LANES
           + jax.lax.broadcasted_iota(jnp.int32, (_ROWS, _LANES), 1))
    valid = (idx < nb) & (score >= _SCORE_THRESH)
    keys = jnp.where(valid, jax.lax.bitcast_convert_type(score, jnp.int32),
                     jnp.int32(-1))

    def bisect(_, carry):
        lo, hi = carry
        mid = (lo + hi) // 2
        cnt = jnp.sum(jnp.where(keys >= mid, 1, 0).astype(jnp.int32))
        big = cnt >= _PRE
        return (jnp.where(big, mid, lo), jnp.where(big, hi, mid))

    k_thr, _ = jax.lax.fori_loop(
        0, 31, bisect, (jnp.int32(0), jnp.int32(0x3F800000)))

    c_gt = jnp.sum(jnp.where(keys >= k_thr + 1, 1, 0).astype(jnp.int32))
    quota = _PRE - c_gt
    eq = (keys == k_thr) & valid
    eq_pfx = _row_major_excl_prefix(eq.astype(jnp.int32))
    sel = (keys > k_thr) | (eq & (eq_pfx < quota))
    sel_i = sel.astype(jnp.int32)
    n_sel = jnp.sum(sel_i)
    pos_ref[...] = jnp.where(sel, _row_major_excl_prefix(sel_i), jnp.int32(-1))

    slot = jax.lax.broadcasted_iota(jnp.int32, (2 * 128, 1), 0)
    acc_ref[...] = jnp.zeros((9, _PRE + 256), jnp.float32)

    def gather_chunk(c, base):
        for k in range(4):
            data = jnp.reshape(box_ref[0, :, pl.ds(c * 4 + k, 1), :],
                               (9, _LANES))
            p = pos_ref[pl.ds(c * 4 + k, 1), :]
            ba = base // 128
            onehot = (slot == (p - ba * 128)).astype(jnp.float32)
            contrib = _DOT(data, onehot, (((1,), (1,)), ((), ())))
            acc_ref[:, pl.ds(ba * 128, 256)] += contrib
            base = base + jnp.sum((p >= 0).astype(jnp.int32))
        return base

    jax.lax.fori_loop(0, _ROWS // 4, gather_chunk, jnp.int32(0))
    compact = acc_ref[:, : _PRE]

    k_row = compact[7:8, :]
    k_col = k_row.T
    col_idx = jax.lax.broadcasted_iota(jnp.int32, (1, _PRE), 1)
    iota128c = jax.lax.broadcasted_iota(jnp.int32, (128, 1), 0)
    rank_parts = []
    for rb in range(_NBLK):
        lo = rb * 128
        krb = k_col[lo : lo + 128, :]
        row_idx = lo + iota128c
        cnt = jnp.zeros((128, 1), jnp.int32)
        for kt in range(_NBLK):
            cs = slice(kt * 128, kt * 128 + 128)
            m = (k_row[:, cs] > krb) | ((k_row[:, cs] == krb)
                                        & (col_idx[:, cs] < row_idx))
            cnt = cnt + jnp.sum(m.astype(jnp.int32), axis=1, keepdims=True)
        rank_parts.append(cnt)
    rank_col = jnp.concatenate(rank_parts, axis=0)
    rank_row = rank_col.T

    sorted_parts = []
    for rb in range(_NBLK):
        lo = rb * 128
        tgt = lo + iota128c
        acc9 = jnp.zeros((128, 9), jnp.float32)
        for kt in range(_NBLK):
            cs = slice(kt * 128, kt * 128 + 128)
            perm = (rank_row[:, cs] == tgt).astype(jnp.float32)
            acc9 = acc9 + _DOT(perm, compact[:, cs], (((1,), (1,)), ((), ())))
        sorted_parts.append(acc9)
    cand = jnp.concatenate(sorted_parts, axis=0)

    xc, yc = cand[:, 0:1], cand[:, 1:2]
    dxc, dyc = cand[:, 3:4], cand[:, 4:5]
    x1_c, x2_c = xc - dxc * 0.5, xc + dxc * 0.5
    y1_c, y2_c = yc - dyc * 0.5, yc + dyc * 0.5
    area_c = dxc * dyc
    x1_r, x2_r = x1_c.T, x2_c.T
    y1_r, y2_r = y1_c.T, y2_c.T
    area_r = area_c.T

    lane2k = jax.lax.broadcasted_iota(jnp.int32, (1, _PRE), 1)
    ut_mask = (jax.lax.broadcasted_iota(jnp.int32, (128, 128), 1)
               > jax.lax.broadcasted_iota(jnp.int32, (128, 128), 0)
               ).astype(jnp.float32)
    keep = (lane2k < n_sel).astype(jnp.float32)

    def sup_tile(lo, jc):
        rs, cs = slice(lo, lo + 128), slice(jc, jc + 128)
        ix1 = jnp.maximum(x1_c[rs, :], x1_r[:, cs])
        ix2 = jnp.minimum(x2_c[rs, :], x2_r[:, cs])
        iy1 = jnp.maximum(y1_c[rs, :], y1_r[:, cs])
        iy2 = jnp.minimum(y2_c[rs, :], y2_r[:, cs])
        inter = (jnp.clip(ix2 - ix1, 0.0, None) *
                 jnp.clip(iy2 - iy1, 0.0, None))
        union = area_c[rs, :] + area_r[:, cs] - inter
        iou = inter / jnp.maximum(union, 1e-6)
        return (iou > _NMS_THRESH).astype(jnp.float32)

    for blk in range(_NBLK):
        lo = blk * 128

        s_blk = sup_tile(lo, lo) * ut_mask
        valid_blk = keep[:, lo : lo + 128]

        def fix_cond(carry):
            return carry[1]

        def fix_body(carry):
            kb, _ = carry
            cnt = _DOT(kb, s_blk, (((1,), (0,)), ((), ())))
            new = valid_blk * (cnt <= 0.5).astype(jnp.float32)
            return (new, jnp.any(new != kb))

        keep_blk, _ = jax.lax.while_loop(
            fix_cond, fix_body, (valid_blk, jnp.bool_(True)))

        parts = ([keep[:, :lo]] if lo else []) + [keep_blk]
        for jb in range(blk + 1, _NBLK):
            jc = jb * 128
            sup_cnt = _DOT(keep_blk, sup_tile(lo, jc),
                           (((1,), (0,)), ((), ())))
            parts.append(
                jnp.where(sup_cnt > 0.5, 0.0, keep[:, jc : jc + 128]))
        keep = jnp.concatenate(parts, axis=1) if len(parts) > 1 else keep_blk

    incl = keep
    sh = 1
    while sh < _PRE:
        incl = incl + jnp.concatenate(
            [jnp.zeros((1, sh), jnp.float32), incl[:, : _PRE - sh]], axis=1)
        sh *= 2
    p2 = jnp.where(keep > 0.5, (incl - keep).astype(jnp.int32), jnp.int32(-1))
    n_kept = jnp.sum(keep).astype(jnp.int32)

    slot_out = jax.lax.broadcasted_iota(jnp.int32, (_POST_PAD, 1), 0)
    out = jnp.zeros((_POST_PAD, 9), jnp.float32)
    for kt in range(_NBLK):
        cs = slice(kt * 128, kt * 128 + 128)
        oh = (slot_out == p2[:, cs]).astype(jnp.float32)
        out = out + _DOT(oh, cand[cs, :], (((1,), (0,)), ((), ())))
    label = out[:, 8:9] + (slot_out < n_kept).astype(jnp.float32)
    out_ref[0, :, :] = jnp.concatenate([out[:, :8], label], axis=1)


def kernel(output_boxes, num_boxes):
    bsz, n, _ = output_boxes.shape
    arr = output_boxes[:, :, jnp.array([0, 1, 2, 3, 4, 5, 6, 8, 7])]
    pad = _ROWS * _LANES - n
    arr = jnp.pad(arr, ((0, 0), (0, pad), (0, 0)))
    arr = jnp.transpose(arr, (0, 2, 1)).reshape(bsz, 9, _ROWS, _LANES)
    nb = num_boxes.astype(jnp.int32)

    padded = pl.pallas_call(
        _nms_kernel,
        grid=(bsz,),
        in_specs=[
            pl.BlockSpec(memory_space=pltpu.SMEM),
            pl.BlockSpec((1, 9, _ROWS, _LANES), lambda b: (b, 0, 0, 0)),
        ],
        out_specs=pl.BlockSpec((1, _POST_PAD, 9), lambda b: (b, 0, 0)),
        out_shape=jax.ShapeDtypeStruct((bsz, _POST_PAD, 9), jnp.float32),
        scratch_shapes=[
            pltpu.VMEM((_ROWS, _LANES), jnp.int32),
            pltpu.VMEM((9, _PRE + 256), jnp.float32),
        ],
        compiler_params=pltpu.CompilerParams(
            dimension_semantics=("parallel",)),
    )(nb, arr)
    return padded[:, :_POST, :]


# speedup vs baseline: 6.9414x; 1.0608x over previous
import functools

import jax
import jax.numpy as jnp
from jax.experimental import pallas as pl
from jax.experimental.pallas import tpu as pltpu

_SCORE_THRESH = 0.1
_NMS_THRESH = 0.7
_PRE = 2048
_POST = 500
_POST_PAD = 512
_ROWS = 160
_LANES = 128
_NBLK = _PRE // 128

_DOT = functools.partial(
    jax.lax.dot_general,
    precision=jax.lax.Precision.HIGHEST,
    preferred_element_type=jnp.float32,
)


def _row_major_excl_prefix(m):
    r, l = m.shape
    incl = m
    sh = 1
    while sh < l:
        incl = incl + jnp.concatenate(
            [jnp.zeros((r, sh), jnp.int32), incl[:, : l - sh]], axis=1)
        sh *= 2
    row_tot = incl[:, l - 1 : l]
    rows_incl = row_tot
    sh = 1
    while sh < r:
        rows_incl = rows_incl + jnp.concatenate(
            [jnp.zeros((sh, 1), jnp.int32), rows_incl[: r - sh, :]], axis=0)
        sh *= 2
    rows_excl = rows_incl - row_tot
    return rows_excl + (incl - m)


def _nms_kernel(nb_ref, box_ref, out_ref, pos_ref, acc_ref):
    b = pl.program_id(0)
    nb = nb_ref[b]

    score = box_ref[0, 7, :, :]
    idx = (jax.lax.broadcasted_iota(jnp.int32, (_ROWS, _LANES), 0) * _LANES
           + jax.lax.broadcasted_iota(jnp.int32, (_ROWS, _LANES), 1))
    valid = (idx < nb) & (score >= _SCORE_THRESH)
    keys = jnp.where(valid, jax.lax.bitcast_convert_type(score, jnp.int32),
                     jnp.int32(-1))

    def bisect(_, carry):
        lo, hi = carry
        mid = (lo + hi) // 2
        cnt = jnp.sum(jnp.where(keys >= mid, 1, 0).astype(jnp.int32))
        big = cnt >= _PRE
        return (jnp.where(big, mid, lo), jnp.where(big, hi, mid))

    k_thr, _ = jax.lax.fori_loop(
        0, 25, bisect, (jnp.int32(0x3DCCCCCC), jnp.int32(0x3F800000)))

    c_gt = jnp.sum(jnp.where(keys >= k_thr + 1, 1, 0).astype(jnp.int32))
    quota = _PRE - c_gt
    eq = (keys == k_thr) & valid
    eq_pfx = _row_major_excl_prefix(eq.astype(jnp.int32))
    sel = (keys > k_thr) | (eq & (eq_pfx < quota))
    sel_i = sel.astype(jnp.int32)
    n_sel = jnp.sum(sel_i)
    pos_ref[...] = jnp.where(sel, _row_major_excl_prefix(sel_i), jnp.int32(-1))

    slot = jax.lax.broadcasted_iota(jnp.int32, (2 * 128, 1), 0)
    acc_ref[...] = jnp.zeros((9, _PRE + 256), jnp.float32)

    def gather_chunk(c, base):
        for k in range(8):
            data = jnp.reshape(box_ref[0, :, pl.ds(c * 8 + k, 1), :],
                               (9, _LANES))
            p = pos_ref[pl.ds(c * 8 + k, 1), :]
            ba = base // 128
            onehot = (slot == (p - ba * 128)).astype(jnp.float32)
            contrib = _DOT(data, onehot, (((1,), (1,)), ((), ())))
            acc_ref[:, pl.ds(ba * 128, 256)] += contrib
            base = base + jnp.sum((p >= 0).astype(jnp.int32))
        return base

    jax.lax.fori_loop(0, _ROWS // 8, gather_chunk, jnp.int32(0))
    compact = acc_ref[:, : _PRE]

    k_row = compact[7:8, :]
    k_col = k_row.T
    col_idx = jax.lax.broadcasted_iota(jnp.int32, (1, _PRE), 1)
    iota128c = jax.lax.broadcasted_iota(jnp.int32, (128, 1), 0)
    rank_parts = []
    for rb in range(_NBLK):
        lo = rb * 128
        krb = k_col[lo : lo + 128, :]
        row_idx = lo + iota128c
        cnt = jnp.zeros((128, 1), jnp.int32)
        for kt in range(_NBLK):
            cs = slice(kt * 128, kt * 128 + 128)
            m = (k_row[:, cs] > krb) | ((k_row[:, cs] == krb)
                                        & (col_idx[:, cs] < row_idx))
            cnt = cnt + jnp.sum(m.astype(jnp.int32), axis=1, keepdims=True)
        rank_parts.append(cnt)
    rank_col = jnp.concatenate(rank_parts, axis=0)
    rank_row = rank_col.T

    sorted_parts = []
    for rb in range(_NBLK):
        lo = rb * 128
        tgt = lo + iota128c
        acc9 = jnp.zeros((128, 9), jnp.float32)
        for kt in range(_NBLK):
            cs = slice(kt * 128, kt * 128 + 128)
            perm = (rank_row[:, cs] == tgt).astype(jnp.float32)
            acc9 = acc9 + _DOT(perm, compact[:, cs], (((1,), (1,)), ((), ())))
        sorted_parts.append(acc9)
    cand = jnp.concatenate(sorted_parts, axis=0)

    xc, yc = cand[:, 0:1], cand[:, 1:2]
    dxc, dyc = cand[:, 3:4], cand[:, 4:5]
    x1_c, x2_c = xc - dxc * 0.5, xc + dxc * 0.5
    y1_c, y2_c = yc - dyc * 0.5, yc + dyc * 0.5
    area_c = dxc * dyc
    x1_r, x2_r = x1_c.T, x2_c.T
    y1_r, y2_r = y1_c.T, y2_c.T
    area_r = area_c.T

    lane2k = jax.lax.broadcasted_iota(jnp.int32, (1, _PRE), 1)
    ut_mask = (jax.lax.broadcasted_iota(jnp.int32, (128, 128), 1)
               > jax.lax.broadcasted_iota(jnp.int32, (128, 128), 0)
               ).astype(jnp.float32)
    keep = (lane2k < n_sel).astype(jnp.float32)

    def sup_tile(lo, jc):
        rs, cs = slice(lo, lo + 128), slice(jc, jc + 128)
        ix1 = jnp.maximum(x1_c[rs, :], x1_r[:, cs])
        ix2 = jnp.minimum(x2_c[rs, :], x2_r[:, cs])
        iy1 = jnp.maximum(y1_c[rs, :], y1_r[:, cs])
        iy2 = jnp.minimum(y2_c[rs, :], y2_r[:, cs])
        inter = (jnp.clip(ix2 - ix1, 0.0, None) *
                 jnp.clip(iy2 - iy1, 0.0, None))
        union = area_c[rs, :] + area_r[:, cs] - inter
        iou = inter / jnp.maximum(union, 1e-6)
        return (iou > _NMS_THRESH).astype(jnp.float32)

    for blk in range(_NBLK):
        lo = blk * 128

        s_blk = sup_tile(lo, lo) * ut_mask
        valid_blk = keep[:, lo : lo + 128]

        def fix_cond(carry):
            return carry[1]

        def fix_body(carry):
            kb, _ = carry
            cnt = _DOT(kb, s_blk, (((1,), (0,)), ((), ())))
            new = valid_blk * (cnt <= 0.5).astype(jnp.float32)
            return (new, jnp.any(new != kb))

        keep_blk, _ = jax.lax.while_loop(
            fix_cond, fix_body, (valid_blk, jnp.bool_(True)))

        parts = ([keep[:, :lo]] if lo else []) + [keep_blk]
        for jb in range(blk + 1, _NBLK):
            jc = jb * 128
            sup_cnt = _DOT(keep_blk, sup_tile(lo, jc),
                           (((1,), (0,)), ((), ())))
            parts.append(
                jnp.where(sup_cnt > 0.5, 0.0, keep[:, jc : jc + 128]))
        keep = jnp.concatenate(parts, axis=1) if len(parts) > 1 else keep_blk

    incl = keep
    sh = 1
    while sh < _PRE:
        incl = incl + jnp.concatenate(
            [jnp.zeros((1, sh), jnp.float32), incl[:, : _PRE - sh]], axis=1)
        sh *= 2
    p2 = jnp.where(keep > 0.5, (incl - keep).astype(jnp.int32), jnp.int32(-1))
    n_kept = jnp.sum(keep).astype(jnp.int32)

    slot_out = jax.lax.broadcasted_iota(jnp.int32, (_POST_PAD, 1), 0)
    out = jnp.zeros((_POST_PAD, 9), jnp.float32)
    for kt in range(_NBLK):
        cs = slice(kt * 128, kt * 128 + 128)
        oh = (slot_out == p2[:, cs]).astype(jnp.float32)
        out = out + _DOT(oh, cand[cs, :], (((1,), (0,)), ((), ())))
    label = out[:, 8:9] + (slot_out < n_kept).astype(jnp.float32)
    out_ref[0, :, :] = jnp.concatenate([out[:, :8], label], axis=1)


def kernel(output_boxes, num_boxes):
    bsz, n, _ = output_boxes.shape
    arr = output_boxes[:, :, jnp.array([0, 1, 2, 3, 4, 5, 6, 8, 7])]
    pad = _ROWS * _LANES - n
    arr = jnp.pad(arr, ((0, 0), (0, pad), (0, 0)))
    arr = jnp.transpose(arr, (0, 2, 1)).reshape(bsz, 9, _ROWS, _LANES)
    nb = num_boxes.astype(jnp.int32)

    padded = pl.pallas_call(
        _nms_kernel,
        grid=(bsz,),
        in_specs=[
            pl.BlockSpec(memory_space=pltpu.SMEM),
            pl.BlockSpec((1, 9, _ROWS, _LANES), lambda b: (b, 0, 0, 0)),
        ],
        out_specs=pl.BlockSpec((1, _POST_PAD, 9), lambda b: (b, 0, 0)),
        out_shape=jax.ShapeDtypeStruct((bsz, _POST_PAD, 9), jnp.float32),
        scratch_shapes=[
            pltpu.VMEM((_ROWS, _LANES), jnp.int32),
            pltpu.VMEM((9, _PRE + 256), jnp.float32),
        ],
        compiler_params=pltpu.CompilerParams(
            dimension_semantics=("parallel",)),
    )(nb, arr)
    return padded[:, :_POST, :]
